# baseline (speedup 1.0000x reference)
import jax
import jax.numpy as jnp
from jax.experimental import pallas as pl


def _linear_kernel(x_ref, w_ref, b_ref, h_ref):
    h_ref[...] = (
        jax.lax.dot_general(
            x_ref[...],
            w_ref[...],
            (((1,), (1,)), ((), ())),
            preferred_element_type=jnp.float32,
        )
        + b_ref[...]
    )


def _agg_kernel(adj_ref, h_ref, out_ref):
    acc = jnp.dot(adj_ref[...], h_ref[...], preferred_element_type=jnp.float32)
    out_ref[...] = jnp.where(acc > 0.0, acc, jnp.exp(acc) - 1.0)


def _pick_block_rows(m: int) -> int:
    for cand in (400, 512, 256, 200, 128, 80, 40, 16, 8):
        if m % cand == 0:
            return cand
    return min(m, 256)


def kernel(x, adj, W, b):
    n, d_in = x.shape
    d_out = W.shape[0]
    m = adj.shape[0]

    b2 = b.reshape(1, d_out).astype(jnp.float32)

    h = pl.pallas_call(
        _linear_kernel,
        out_shape=jax.ShapeDtypeStruct((n, d_out), jnp.float32),
    )(x, W, b2)

    bm = _pick_block_rows(m)
    out = pl.pallas_call(
        _agg_kernel,
        grid=(pl.cdiv(m, bm),),
        in_specs=[
            pl.BlockSpec((bm, n), lambda i: (i, 0)),
            pl.BlockSpec((n, d_out), lambda i: (0, 0)),
        ],
        out_specs=pl.BlockSpec((bm, d_out), lambda i: (i, 0)),
        out_shape=jax.ShapeDtypeStruct((m, d_out), jnp.float32),
    )(adj, h)
    return out
